# baseline (speedup 1.0000x reference)
import jax
import jax.numpy as jnp
from jax.experimental import pallas as pl
from jax.experimental.pallas import tpu as pltpu

M = 32768
D_MODEL = 768
NUM_EXPERTS = 64
CLAMP_MIN = -10000.0
CLAMP_MAX = 10000.0

BLOCK_M = 1024


def _router_kernel(x_ref, wt_ref, b_ref, out_ref):
    raw = jnp.dot(x_ref[...], wt_ref[...], preferred_element_type=jnp.float32)
    raw = raw + b_ref[...]
    out_ref[...] = jnp.clip(raw, CLAMP_MIN, CLAMP_MAX)


def kernel(x, W, expert_bias):
    wt = W.T
    bias = expert_bias.reshape(1, NUM_EXPERTS)
    grid = (M // BLOCK_M,)
    logits = pl.pallas_call(
        _router_kernel,
        grid=grid,
        in_specs=[
            pl.BlockSpec((BLOCK_M, D_MODEL), lambda i: (i, 0)),
            pl.BlockSpec((D_MODEL, NUM_EXPERTS), lambda i: (0, 0)),
            pl.BlockSpec((1, NUM_EXPERTS), lambda i: (0, 0)),
        ],
        out_specs=pl.BlockSpec((BLOCK_M, NUM_EXPERTS), lambda i: (i, 0)),
        out_shape=jax.ShapeDtypeStruct((M, NUM_EXPERTS), jnp.float32),
        compiler_params=pltpu.CompilerParams(
            dimension_semantics=("arbitrary",),
        ),
    )(x, wt, bias)
    return (logits, logits)
